# Initial kernel scaffold; baseline (speedup 1.0000x reference)
import jax
import jax.numpy as jnp
from jax.experimental import pallas as pl


def kernel(x, w1, b1, w2, b2, w3, b3, w4, b4, w5, b5, w6, b6, w7, b7, w8, b8, w9, b9, l1w, l1b, l2w, l2b, A):
    raise NotImplementedError("write your pallas kernel here")


# speedup vs baseline: 9.7314x; 9.7314x over previous
import jax
import jax.numpy as jnp
from jax import lax
from jax.experimental import pallas as pl
from jax.experimental.pallas import tpu as pltpu

_NB = 256


def _lk(v):
    return jnp.maximum(v, 0.2 * v)


def _shr(x):
    z = jnp.zeros_like(x[:, :1])
    return jnp.concatenate([z, x[:, :-1]], axis=1)


def _shl(x):
    z = jnp.zeros_like(x[:, :1])
    return jnp.concatenate([x[:, 1:], z], axis=1)


def _conv_vpu(planes, wr, br, co, ci):
    ms = [_shr(p) for p in planes]
    ps = [_shl(p) for p in planes]
    out = []
    for k in range(co):
        acc = None
        for c in range(ci):
            t = wr[k, c, 0] * ms[c] + wr[k, c, 1] * planes[c] + wr[k, c, 2] * ps[c]
            acc = t if acc is None else acc + t
        out.append(_lk(acc + br[k]))
    return out


def _dec_kernel(a_ref, w6_ref, b6_ref, w7_ref, b7_ref, w8_ref, b8_ref,
                p2_ref, out_ref):
    hi = jax.lax.Precision.HIGHEST
    a = a_ref[...]
    am, ap = _shr(a), _shl(a)
    h = (w6_ref[:, 0:1] * am + w6_ref[:, 1:2] * a + w6_ref[:, 2:3] * ap
         + b6_ref[...])
    h = _lk(h)
    s = jnp.concatenate([_shr(h), h, _shl(h)], axis=0)
    h = _lk(jnp.dot(w7_ref[...], s, precision=hi,
                    preferred_element_type=jnp.float32) + b7_ref[...])
    h = _lk(jnp.dot(w8_ref[...], h, precision=hi,
                    preferred_element_type=jnp.float32) + b8_ref[...])
    out_ref[...] = jnp.dot(p2_ref[...], h, precision=hi,
                           preferred_element_type=jnp.float32)


def _main_kernel(x_ref, w1r, b1r, w2r, b2r, w3r, b3r, w4r, b4r, w5r, b5r,
                 w9r, b9r, p1_ref, pga_ref, l1w_ref, l1b_ref, l2w_ref,
                 l2b_ref, out_ref):
    x = x_ref[:, 0, :]
    h1 = _conv_vpu([x], w1r, b1r, 4, 1)
    h2 = _conv_vpu(h1, w2r, b2r, 8, 4)
    hcat = jnp.concatenate([h.astype(jnp.bfloat16) for h in h2], axis=0)
    hp = jnp.dot(hcat, p1_ref[...],
                 preferred_element_type=jnp.float32)
    g = [hp[k * _NB:(k + 1) * _NB, :] for k in range(8)]
    g = _conv_vpu(g, w3r, b3r, 4, 8)
    g = _conv_vpu(g, w4r, b4r, 4, 4)
    g = _conv_vpu(g, w5r, b5r, 4, 4)
    gcat = jnp.concatenate([h.astype(jnp.bfloat16) for h in g], axis=0)
    gg = jnp.dot(gcat, pga_ref[...],
                 preferred_element_type=jnp.float32)
    gc = [gg[c * _NB:(c + 1) * _NB, :] for c in range(4)]
    h9 = _conv_vpu(gc, w9r, b9r, 1, 4)[0]
    z = jnp.dot(h9.astype(jnp.bfloat16), l1w_ref[...],
                preferred_element_type=jnp.float32) + l1b_ref[...]
    z = _lk(z)
    out_ref[...] = jnp.dot(z.astype(jnp.bfloat16), l2w_ref[...],
                           preferred_element_type=jnp.float32) + l2b_ref[...]


def kernel(x, w1, b1, w2, b2, w3, b3, w4, b4, w5, b5, w6, b6, w7, b7, w8, b8,
           w9, b9, l1w, l1b, l2w, l2b, A):
    n, cin, L = x.shape
    assert cin == 1

    p1 = (0.25 * jnp.kron(jnp.eye(L // 4, dtype=jnp.float32),
                          jnp.ones((4, 1), jnp.float32)))
    p2 = (0.25 * jnp.kron(jnp.eye(32, dtype=jnp.float32),
                          jnp.ones((4, 1), jnp.float32)))

    w7f = jnp.transpose(w7, (0, 2, 1)).reshape(32, 48)
    pga = pl.pallas_call(
        _dec_kernel,
        out_shape=jax.ShapeDtypeStruct((128, 32), jnp.float32),
    )(A.reshape(1, 32), w6[:, 0, :], b6[:, None], w7f, b7[:, None],
      w8[:, :, 0], b8[:, None], p2)

    n_pad = (-n) % _NB
    if n_pad:
        x = jnp.concatenate([x, jnp.zeros((n_pad, 1, L), x.dtype)], axis=0)
    n_tot = n + n_pad

    l2w_p = jnp.zeros((16, 8), jnp.float32).at[:, :3].set(l2w.T)
    l2b_p = jnp.zeros((1, 8), jnp.float32).at[:, :3].set(l2b[None, :])

    smem = [w1, b1, w2, b2, w3, b3, w4, b4, w5, b5, w9, b9]
    vmem = [p1.astype(jnp.bfloat16), pga.astype(jnp.bfloat16),
            l1w.T.astype(jnp.bfloat16), l1b[None, :],
            l2w_p.astype(jnp.bfloat16), l2b_p]

    def smem_spec(a):
        return pl.BlockSpec(memory_space=pltpu.SMEM)

    def vmem_spec(a):
        nd = a.ndim
        return pl.BlockSpec(a.shape, lambda i, _nd=nd: (0,) * _nd)

    out = pl.pallas_call(
        _main_kernel,
        out_shape=jax.ShapeDtypeStruct((n_tot, 8), jnp.float32),
        grid_spec=pltpu.PrefetchScalarGridSpec(
            num_scalar_prefetch=0,
            grid=(n_tot // _NB,),
            in_specs=([pl.BlockSpec((_NB, 1, L), lambda i: (i, 0, 0))]
                      + [smem_spec(a) for a in smem]
                      + [vmem_spec(a) for a in vmem]),
            out_specs=pl.BlockSpec((_NB, 8), lambda i: (i, 0)),
        ),
        compiler_params=pltpu.CompilerParams(
            dimension_semantics=("parallel",)),
    )(x, *smem, *vmem)

    return out[:n, :3]


# speedup vs baseline: 9.7572x; 1.0027x over previous
import jax
import jax.numpy as jnp
from jax import lax
from jax.experimental import pallas as pl
from jax.experimental.pallas import tpu as pltpu

_NB = 256


def _lk(v):
    return jnp.maximum(v, 0.2 * v)


def _shr(x):
    z = jnp.zeros_like(x[:, :1])
    return jnp.concatenate([z, x[:, :-1]], axis=1)


def _shl(x):
    z = jnp.zeros_like(x[:, :1])
    return jnp.concatenate([x[:, 1:], z], axis=1)


def _conv_vpu(planes, wr, br, co, ci, dtype=jnp.float32):
    def w(k, c, t):
        return wr[k, c, t].astype(dtype)

    out = []
    if ci <= co:
        ms = [_shr(p) for p in planes]
        ps = [_shl(p) for p in planes]
        for k in range(co):
            acc = None
            for c in range(ci):
                t = w(k, c, 0) * ms[c] + w(k, c, 1) * planes[c] \
                    + w(k, c, 2) * ps[c]
                acc = t if acc is None else acc + t
            out.append(_lk(acc + br[k].astype(dtype)))
    else:
        for k in range(co):
            u = v = z = None
            for c in range(ci):
                tu = w(k, c, 0) * planes[c]
                tv = w(k, c, 1) * planes[c]
                tz = w(k, c, 2) * planes[c]
                u = tu if u is None else u + tu
                v = tv if v is None else v + tv
                z = tz if z is None else z + tz
            acc = _shr(u) + v + _shl(z)
            out.append(_lk(acc + br[k].astype(dtype)))
    return out


def _conv_bf16(planes_f32, wr, br, co, ci):
    bf = jnp.bfloat16
    ms = [_shr(p).astype(bf) for p in planes_f32]
    cs = [p.astype(bf) for p in planes_f32]
    ps = [_shl(p).astype(bf) for p in planes_f32]
    out = []
    for k in range(co):
        acc = None
        for c in range(ci):
            t = (wr[k, c, 0].astype(bf) * ms[c]
                 + wr[k, c, 1].astype(bf) * cs[c]
                 + wr[k, c, 2].astype(bf) * ps[c])
            acc = t if acc is None else acc + t
        out.append(_lk(acc.astype(jnp.float32) + br[k]))
    return out


def _dec_kernel(a_ref, w6_ref, b6_ref, w7_ref, b7_ref, w8_ref, b8_ref,
                p2_ref, out_ref):
    hi = jax.lax.Precision.HIGHEST
    a = a_ref[...]
    am, ap = _shr(a), _shl(a)
    h = (w6_ref[:, 0:1] * am + w6_ref[:, 1:2] * a + w6_ref[:, 2:3] * ap
         + b6_ref[...])
    h = _lk(h)
    s = jnp.concatenate([_shr(h), h, _shl(h)], axis=0)
    h = _lk(jnp.dot(w7_ref[...], s, precision=hi,
                    preferred_element_type=jnp.float32) + b7_ref[...])
    h = _lk(jnp.dot(w8_ref[...], h, precision=hi,
                    preferred_element_type=jnp.float32) + b8_ref[...])
    out_ref[...] = jnp.dot(p2_ref[...], h, precision=hi,
                           preferred_element_type=jnp.float32)


def _main_kernel(x_ref, w1r, b1r, w2r, b2r, w3r, b3r, w4r, b4r, w5r, b5r,
                 w9r, b9r, p1_ref, pga_ref, l1w_ref, l1b_ref, l2w_ref,
                 l2b_ref, out_ref):
    x = x_ref[:, 0, :]
    h1 = _conv_vpu([x], w1r, b1r, 4, 1)
    h2f = _conv_vpu(h1, w2r, b2r, 8, 4)
    h2 = [h.astype(jnp.bfloat16) for h in h2f]
    hcat = jnp.concatenate(h2, axis=0)
    hp = jnp.dot(hcat, p1_ref[...],
                 preferred_element_type=jnp.float32)
    g = [hp[k * _NB:(k + 1) * _NB, :] for k in range(8)]
    g = _conv_vpu(g, w3r, b3r, 4, 8)
    g = _conv_vpu(g, w4r, b4r, 4, 4)
    g = _conv_vpu(g, w5r, b5r, 4, 4)
    gcat = jnp.concatenate([h.astype(jnp.bfloat16) for h in g], axis=0)
    gg = jnp.dot(gcat, pga_ref[...],
                 preferred_element_type=jnp.float32)
    gc = [gg[c * _NB:(c + 1) * _NB, :] for c in range(4)]
    h9 = _conv_vpu(gc, w9r, b9r, 1, 4)[0]
    z = jnp.dot(h9, l1w_ref[...], precision=jax.lax.Precision.HIGHEST,
                preferred_element_type=jnp.float32) + l1b_ref[...]
    z = _lk(z)
    out_ref[...] = jnp.dot(z, l2w_ref[...], precision=jax.lax.Precision.HIGHEST,
                           preferred_element_type=jnp.float32) + l2b_ref[...]


def kernel(x, w1, b1, w2, b2, w3, b3, w4, b4, w5, b5, w6, b6, w7, b7, w8, b8,
           w9, b9, l1w, l1b, l2w, l2b, A):
    n, cin, L = x.shape
    assert cin == 1

    p1 = (0.25 * jnp.kron(jnp.eye(L // 4, dtype=jnp.float32),
                          jnp.ones((4, 1), jnp.float32)))
    p2 = (0.25 * jnp.kron(jnp.eye(32, dtype=jnp.float32),
                          jnp.ones((4, 1), jnp.float32)))

    w7f = jnp.transpose(w7, (0, 2, 1)).reshape(32, 48)
    pga = pl.pallas_call(
        _dec_kernel,
        out_shape=jax.ShapeDtypeStruct((128, 32), jnp.float32),
    )(A.reshape(1, 32), w6[:, 0, :], b6[:, None], w7f, b7[:, None],
      w8[:, :, 0], b8[:, None], p2)

    n_pad = (-n) % _NB
    if n_pad:
        x = jnp.concatenate([x, jnp.zeros((n_pad, 1, L), x.dtype)], axis=0)
    n_tot = n + n_pad

    l2w_p = jnp.zeros((16, 8), jnp.float32).at[:, :3].set(l2w.T)
    l2b_p = jnp.zeros((1, 8), jnp.float32).at[:, :3].set(l2b[None, :])

    smem = [w1, b1, w2, b2, w3, b3, w4, b4, w5, b5, w9, b9]
    vmem = [p1.astype(jnp.bfloat16), pga.astype(jnp.bfloat16),
            l1w.T, l1b[None, :],
            l2w_p, l2b_p]

    def smem_spec(a):
        return pl.BlockSpec(memory_space=pltpu.SMEM)

    def vmem_spec(a):
        nd = a.ndim
        return pl.BlockSpec(a.shape, lambda i, _nd=nd: (0,) * _nd)

    out = pl.pallas_call(
        _main_kernel,
        out_shape=jax.ShapeDtypeStruct((n_tot, 8), jnp.float32),
        grid_spec=pltpu.PrefetchScalarGridSpec(
            num_scalar_prefetch=0,
            grid=(n_tot // _NB,),
            in_specs=([pl.BlockSpec((_NB, 1, L), lambda i: (i, 0, 0))]
                      + [smem_spec(a) for a in smem]
                      + [vmem_spec(a) for a in vmem]),
            out_specs=pl.BlockSpec((_NB, 8), lambda i: (i, 0)),
        ),
        compiler_params=pltpu.CompilerParams(
            dimension_semantics=("parallel",)),
    )(x, *smem, *vmem)

    return out[:n, :3]
